# Initial kernel scaffold; baseline (speedup 1.0000x reference)
import jax
import jax.numpy as jnp
from jax.experimental import pallas as pl


def kernel(ctx_xyz, ctx_dist, qry_xyz, W1, b1, W2, b2, Wq1, bq1, Wq2, bq2, Wd, bd):
    raise NotImplementedError("write your pallas kernel here")


# speedup vs baseline: 2.4650x; 2.4650x over previous
import functools

import jax
import jax.numpy as jnp
from jax import lax
from jax.experimental import pallas as pl
from jax.experimental.pallas import tpu as pltpu
from jax.experimental.pallas import tpu_sc as plsc

RES = 128
C = 64
H = 256
NROWS = RES * RES
ROWW = C + 16
NC, NS = 2, 16
PCH = 32


def _gelu(x):
    return 0.5 * x * (1.0 + lax.erf(x * jnp.float32(0.7071067811865476)))


def _to_pixel(c):
    return (jnp.clip(c, -1.0, 1.0) + 1.0) * 0.5 * float(RES - 1)


def _axis_corners(c):
    px = _to_pixel(c)
    i0 = jnp.clip(jnp.floor(px).astype(jnp.int32), 0, RES - 1)
    i1 = jnp.clip(i0 + 1, 0, RES - 1)
    f = jnp.clip(px - i0.astype(jnp.float32), 0.0, 1.0)
    return i0, i1, f


def _plane_corner_arrays(xyz):
    ax = [_axis_corners(xyz[:, d]) for d in range(3)]
    planes = [(0, 1), (0, 2), (1, 2)]
    out = []
    for (da, db) in planes:
        x0, x1, fx = ax[da]
        y0, y1, fy = ax[db]
        corners = [
            (y0 * RES + x0, (1.0 - fx) * (1.0 - fy)),
            (y0 * RES + x1, fx * (1.0 - fy)),
            (y1 * RES + x0, (1.0 - fx) * fy),
            (y1 * RES + x1, fx * fy),
        ]
        out.append(corners)
    return out


def _pack_chunks(arrs, blk):
    r = [a.reshape(blk // PCH, 1, PCH) for a in arrs]
    return jnp.concatenate(r, axis=1).reshape(blk // PCH, 4 * PCH)


def _expand_x16(w):
    exp = jnp.broadcast_to(w[..., None], (*w.shape, 16))
    return exp.reshape(*w.shape[:-1], w.shape[-1] * 16)


def _ctx_body(xyz_ref, dist_ref, W1_ref, b1_ref, W2_ref, b2_ref,
              feat_ref, cidx_ref, cw_ref):
    blk = xyz_ref.shape[1]
    xyz = xyz_ref[0]
    dist = dist_ref[0]
    x4 = jnp.concatenate([xyz, dist], axis=1)
    h = _gelu(jnp.dot(x4, W1_ref[...],
                      preferred_element_type=jnp.float32) + b1_ref[...])
    feat = jnp.dot(h, W2_ref[...],
                   preferred_element_type=jnp.float32) + b2_ref[...]
    feat_ref[0] = feat

    per_plane = _plane_corner_arrays(xyz)
    for p, corners in enumerate(per_plane):
        cidx_ref[0, p] = _pack_chunks([c[0] for c in corners], blk)
        cw_ref[0, p] = _pack_chunks([c[1] for c in corners], blk)


def _ctx_mlp(ctx_xyz, ctx_dist, W1, b1, W2, b2):
    B, N, _ = ctx_xyz.shape
    blk = 2048
    grid = (B, N // blk)
    nch = N // PCH
    bch = blk // PCH
    return pl.pallas_call(
        _ctx_body,
        grid=grid,
        in_specs=[
            pl.BlockSpec((1, blk, 3), lambda b, j: (b, j, 0)),
            pl.BlockSpec((1, blk, 1), lambda b, j: (b, j, 0)),
            pl.BlockSpec((4, H), lambda b, j: (0, 0)),
            pl.BlockSpec((H,), lambda b, j: (0,)),
            pl.BlockSpec((H, C), lambda b, j: (0, 0)),
            pl.BlockSpec((C,), lambda b, j: (0,)),
        ],
        out_specs=[
            pl.BlockSpec((1, blk, C), lambda b, j: (b, j, 0)),
            pl.BlockSpec((1, 3, bch, 4 * PCH), lambda b, j: (b, 0, j, 0)),
            pl.BlockSpec((1, 3, bch, 4 * PCH), lambda b, j: (b, 0, j, 0)),
        ],
        out_shape=[
            jax.ShapeDtypeStruct((B, N, C), jnp.float32),
            jax.ShapeDtypeStruct((B, 3, nch, 4 * PCH), jnp.int32),
            jax.ShapeDtypeStruct((B, 3, nch, 4 * PCH), jnp.float32),
        ],
    )(ctx_xyz, ctx_dist[..., None], W1, b1, W2, b2)


def _qprep_body(xyz_ref, qidx_ref, qw_ref):
    blk = xyz_ref.shape[0]
    xyz = xyz_ref[...]
    i = pl.program_id(0)
    nblk_per_batch = pl.num_programs(0) // 4
    b = i // nblk_per_batch
    per_plane = _plane_corner_arrays(xyz)
    for p, corners in enumerate(per_plane):
        gofs = (b * 3 + p) * NROWS
        qidx_ref[p] = _pack_chunks([c[0] + gofs for c in corners], blk)
        qw_ref[p] = _pack_chunks([c[1] for c in corners], blk)


def _query_prep(qry_xyz_flat):
    BQ, _ = qry_xyz_flat.shape
    blk = 2048
    grid = (BQ // blk,)
    nch = BQ // PCH
    bch = blk // PCH
    return pl.pallas_call(
        _qprep_body,
        grid=grid,
        in_specs=[pl.BlockSpec((blk, 3), lambda i: (i, 0))],
        out_specs=[
            pl.BlockSpec((3, bch, 4 * PCH), lambda i: (0, i, 0)),
            pl.BlockSpec((3, bch, 4 * PCH), lambda i: (0, i, 0)),
        ],
        out_shape=[
            jax.ShapeDtypeStruct((3, nch, 4 * PCH), jnp.int32),
            jax.ShapeDtypeStruct((3, nch, 4 * PCH), jnp.float32),
        ],
    )(qry_xyz_flat)


def _splat_body(feat_hbm, cidx_hbm, cw_hbm, planes_hbm,
                shared, zbuf, fbuf, ibuf, wxbuf, valbuf, nbuf, obuf):
    cid = lax.axis_index("c")
    sid = lax.axis_index("s")
    N = feat_hbm.shape[1]
    pts_per_tile = N // NS
    nchunk = pts_per_tile // PCH
    rows_per_tile = NROWS // NS
    ZR = zbuf.shape[0]

    def zinit(i, _):
        for t in range(ROWW // 16):
            zbuf[i, pl.ds(16 * t, 16)] = jnp.zeros((16,), jnp.float32)
        return 0
    lax.fori_loop(0, ZR, zinit, 0)

    for kb in range(6):
        b = 2 * cid + (kb // 3)
        p = kb % 3

        for z in range(rows_per_tile // ZR):
            pltpu.sync_copy(
                zbuf, shared.at[pl.ds(sid * rows_per_tile + z * ZR, ZR)])
        plsc.subcore_barrier()

        def chunk_body(j, _):
            pt0 = sid * pts_per_tile + j * PCH
            ch = sid * nchunk + j
            pltpu.sync_copy(feat_hbm.at[b, pl.ds(pt0, PCH)], fbuf)
            pltpu.sync_copy(cidx_hbm.at[b, p, ch], ibuf.at[0])
            pltpu.sync_copy(cw_hbm.at[b, p, ch], wxbuf)

            def q_body(q, _):
                f = [fbuf[q, pl.ds(16 * t, 16)] for t in range(C // 16)]
                for c in range(4):
                    row = c * PCH + q
                    wv = wxbuf[pl.ds(row * 16, 16)]
                    for t in range(C // 16):
                        valbuf[row, pl.ds(16 * t, 16)] = f[t] * wv
                    valbuf[row, pl.ds(C, 16)] = wv
                return 0
            lax.fori_loop(0, PCH, q_body, 0)
            pltpu.sync_copy(valbuf, shared.at[ibuf.at[0]], add=True)
            return 0
        lax.fori_loop(0, nchunk, chunk_body, 0)
        plsc.subcore_barrier()

        NB = nbuf.shape[0]
        def norm_body(r, _):
            row0 = sid * rows_per_tile + r * NB
            pltpu.sync_copy(shared.at[pl.ds(row0, NB)], nbuf)

            def rbody(i, _):
                w = nbuf[i, pl.ds(C, 16)]
                rec = jnp.float32(1.0) / jnp.maximum(w, jnp.float32(1e-6))
                for t in range(C // 16):
                    obuf[i, pl.ds(16 * t, 16)] = nbuf[i, pl.ds(16 * t, 16)] * rec
                return 0
            lax.fori_loop(0, NB, rbody, 0)
            pltpu.sync_copy(obuf, planes_hbm.at[b, p, pl.ds(row0, NB)])
            return 0
        lax.fori_loop(0, rows_per_tile // NB, norm_body, 0)
        plsc.subcore_barrier()


def _splat(ctx_feat, cidx, cw):
    B = ctx_feat.shape[0]
    mesh = plsc.VectorSubcoreMesh(core_axis_name="c", subcore_axis_name="s")
    return pl.kernel(
        _splat_body,
        compiler_params=pltpu.CompilerParams(use_tc_tiling_on_sc=False),
        out_type=jax.ShapeDtypeStruct((B, 3, NROWS, C), jnp.float32),
        mesh=mesh,
        scratch_types=[
            pltpu.VMEM_SHARED((NROWS, ROWW), jnp.float32),
            pltpu.VMEM((32, ROWW), jnp.float32),
            pltpu.VMEM((PCH, C), jnp.float32),
            pltpu.VMEM((1, 4 * PCH), jnp.int32),
            pltpu.VMEM((4 * PCH * 16,), jnp.float32),
            pltpu.VMEM((4 * PCH, ROWW), jnp.float32),
            pltpu.VMEM((64, ROWW), jnp.float32),
            pltpu.VMEM((64, C), jnp.float32),
        ],
    )(ctx_feat, cidx, cw)


def _sample_body(planes_hbm, qidx_hbm, qw_hbm, fused_hbm,
                 ibuf, wxbuf, rbuf, obuf, sem):
    cid = lax.axis_index("c")
    sid = lax.axis_index("s")
    wid = sid * NC + cid
    BQ = fused_hbm.shape[0]
    q_per_tile = BQ // (NC * NS)
    nchunk = q_per_tile // PCH

    def chunk_body(j, _):
        ch = wid * nchunk + j
        for p in range(3):
            pltpu.sync_copy(qidx_hbm.at[p, ch], ibuf.at[p])
            pltpu.sync_copy(qw_hbm.at[p, ch],
                            wxbuf.at[pl.ds(p * 4 * PCH * 16, 4 * PCH * 16)])
        cps = [pltpu.async_copy(planes_hbm.at[ibuf.at[p]], rbuf.at[p], sem)
               for p in range(3)]
        for cp in cps:
            cp.wait()

        def q_body(q, _):
            acc = []
            for p in range(3):
                a = None
                for c in range(4):
                    row = c * PCH + q
                    wv = wxbuf[pl.ds((p * 4 * PCH + row) * 16, 16)]
                    t4 = [rbuf[p, row, pl.ds(16 * t, 16)] * wv
                          for t in range(C // 16)]
                    a = t4 if a is None else [x + y for x, y in zip(a, t4)]
                acc.append(a)
            for t in range(C // 16):
                obuf[q, pl.ds(16 * t, 16)] = (acc[0][t] * acc[1][t]
                                              * acc[2][t])
            return 0
        lax.fori_loop(0, PCH, q_body, 0)
        pltpu.sync_copy(obuf, fused_hbm.at[pl.ds(wid * q_per_tile + j * PCH,
                                                 PCH)])
        return 0
    lax.fori_loop(0, nchunk, chunk_body, 0)


def _sample(planes_flat, qidx, qw, BQ):
    mesh = plsc.VectorSubcoreMesh(core_axis_name="c", subcore_axis_name="s")
    return pl.kernel(
        _sample_body,
        compiler_params=pltpu.CompilerParams(use_tc_tiling_on_sc=False),
        out_type=jax.ShapeDtypeStruct((BQ, C), jnp.float32),
        mesh=mesh,
        scratch_types=[
            pltpu.VMEM((3, 4 * PCH), jnp.int32),
            pltpu.VMEM((3 * 4 * PCH * 16,), jnp.float32),
            pltpu.VMEM((3, 4 * PCH, C), jnp.float32),
            pltpu.VMEM((PCH, C), jnp.float32),
            pltpu.SemaphoreType.DMA,
        ],
    )(planes_flat, qidx, qw)


def _qmlp_body(fused_ref, Wq1_ref, bq1_ref, Wq2_ref, bq2_ref, Wd_ref, bd_ref,
               out_ref):
    x = fused_ref[...]
    h1 = _gelu(jnp.dot(x, Wq1_ref[...],
                       preferred_element_type=jnp.float32) + bq1_ref[...])
    h2 = _gelu(jnp.dot(h1, Wq2_ref[...],
                       preferred_element_type=jnp.float32) + bq2_ref[...])
    o = jnp.sum(h2 * Wd_ref[...].reshape(1, H), axis=1, keepdims=True)
    out_ref[...] = o + bd_ref[...]


def _query_mlp(fused, Wq1, bq1, Wq2, bq2, Wd, bd):
    BQ = fused.shape[0]
    blk = 2048
    return pl.pallas_call(
        _qmlp_body,
        grid=(BQ // blk,),
        in_specs=[
            pl.BlockSpec((blk, C), lambda i: (i, 0)),
            pl.BlockSpec((C, H), lambda i: (0, 0)),
            pl.BlockSpec((H,), lambda i: (0,)),
            pl.BlockSpec((H, H), lambda i: (0, 0)),
            pl.BlockSpec((H,), lambda i: (0,)),
            pl.BlockSpec((H, 1), lambda i: (0, 0)),
            pl.BlockSpec((1,), lambda i: (0,)),
        ],
        out_specs=pl.BlockSpec((blk, 1), lambda i: (i, 0)),
        out_shape=jax.ShapeDtypeStruct((BQ, 1), jnp.float32),
    )(fused, Wq1, bq1, Wq2, bq2, Wd, bd)


def kernel(ctx_xyz, ctx_dist, qry_xyz, W1, b1, W2, b2, Wq1, bq1, Wq2, bq2,
           Wd, bd):
    B, N, _ = ctx_xyz.shape
    Q = qry_xyz.shape[1]
    BQ = B * Q

    ctx_feat, cidx, cw = _ctx_mlp(ctx_xyz, ctx_dist, W1, b1, W2, b2)
    planes = _splat(ctx_feat, cidx, _expand_x16(cw))
    qidx, qw = _query_prep(qry_xyz.reshape(BQ, 3))
    fused = _sample(planes.reshape(B * 3 * NROWS, C), qidx,
                    _expand_x16(qw), BQ)
    out = _query_mlp(fused, Wq1, bq1, Wq2, bq2, Wd, bd)
    return out.reshape(B, Q, 1)


# speedup vs baseline: 4.5711x; 1.8544x over previous
import jax
import jax.numpy as jnp
from jax import lax
from jax.experimental import pallas as pl
from jax.experimental.pallas import tpu as pltpu
from jax.experimental.pallas import tpu_sc as plsc

RES = 128
C = 64
H = 256
NROWS = RES * RES
ROWW = C + 16
NC, NS = 2, 16
PCH = 32
QCH = 64

_SC_PARAMS = pltpu.CompilerParams(use_tc_tiling_on_sc=False,
                                  needs_layout_passes=False)


def _gelu(x):
    return 0.5 * x * (1.0 + lax.erf(x * jnp.float32(0.7071067811865476)))


def _to_pixel(c):
    return (jnp.clip(c, -1.0, 1.0) + 1.0) * 0.5 * float(RES - 1)


def _axis_corners(c):
    px = _to_pixel(c)
    i0 = jnp.clip(jnp.floor(px).astype(jnp.int32), 0, RES - 1)
    i1 = jnp.clip(i0 + 1, 0, RES - 1)
    f = jnp.clip(px - i0.astype(jnp.float32), 0.0, 1.0)
    return i0, i1, f


def _plane_corners_rows(xyzT):
    ax = [_axis_corners(xyzT[d:d + 1, :]) for d in range(3)]
    out = []
    for (da, db) in ((0, 1), (0, 2), (1, 2)):
        x0, x1, fx = ax[da]
        y0, y1, fy = ax[db]
        out.append([
            (y0 * RES + x0, (1.0 - fx) * (1.0 - fy)),
            (y0 * RES + x1, fx * (1.0 - fy)),
            (y1 * RES + x0, (1.0 - fx) * fy),
            (y1 * RES + x1, fx * fy),
        ])
    return out


def _lane_bcast(vec16, lane):
    idx = jnp.full((16,), lane, jnp.int32)
    return vec16.at[idx].get(mode="promise_in_bounds")


def _ctx_body(xyz_ref, dist_ref, xyzT_ref, W1_ref, b1_ref, W2_ref, b2_ref,
              feat_ref, cidx_ref, cw_ref):
    blk = xyz_ref.shape[1]
    xyz = xyz_ref[0]
    dist = dist_ref[0]
    x4 = jnp.concatenate([xyz, dist], axis=1)
    h = _gelu(jnp.dot(x4, W1_ref[...],
                      preferred_element_type=jnp.float32) + b1_ref[...])
    feat = jnp.dot(h, W2_ref[...],
                   preferred_element_type=jnp.float32) + b2_ref[...]
    feat_ref[0] = feat

    per_plane = _plane_corners_rows(xyzT_ref[0])
    for p, corners in enumerate(per_plane):
        for c, (idx, w) in enumerate(corners):
            cidx_ref[0, p, pl.ds(c, 1)] = idx
            cw_ref[0, p, pl.ds(c, 1)] = w


def _ctx_mlp(ctx_xyz, ctx_dist, W1, b1, W2, b2):
    B, N, _ = ctx_xyz.shape
    blk = 2048
    grid = (B, N // blk)
    xyzT = ctx_xyz.transpose(0, 2, 1)
    return pl.pallas_call(
        _ctx_body,
        grid=grid,
        in_specs=[
            pl.BlockSpec((1, blk, 3), lambda b, j: (b, j, 0)),
            pl.BlockSpec((1, blk, 1), lambda b, j: (b, j, 0)),
            pl.BlockSpec((1, 3, blk), lambda b, j: (b, 0, j)),
            pl.BlockSpec((4, H), lambda b, j: (0, 0)),
            pl.BlockSpec((H,), lambda b, j: (0,)),
            pl.BlockSpec((H, C), lambda b, j: (0, 0)),
            pl.BlockSpec((C,), lambda b, j: (0,)),
        ],
        out_specs=[
            pl.BlockSpec((1, blk, C), lambda b, j: (b, j, 0)),
            pl.BlockSpec((1, 3, 4, blk), lambda b, j: (b, 0, 0, j)),
            pl.BlockSpec((1, 3, 4, blk), lambda b, j: (b, 0, 0, j)),
        ],
        out_shape=[
            jax.ShapeDtypeStruct((B, N, C), jnp.float32),
            jax.ShapeDtypeStruct((B, 3, 4, N), jnp.int32),
            jax.ShapeDtypeStruct((B, 3, 4, N), jnp.float32),
        ],
    )(ctx_xyz, ctx_dist[..., None], xyzT, W1, b1, W2, b2)


def _qprep_body(xyzT_ref, qidx_ref, qw_ref):
    i = pl.program_id(0)
    nblk_per_batch = pl.num_programs(0) // 4
    b = i // nblk_per_batch
    per_plane = _plane_corners_rows(xyzT_ref[...])
    for p, corners in enumerate(per_plane):
        gofs = (b * 3 + p) * NROWS
        for c, (idx, w) in enumerate(corners):
            qidx_ref[p, pl.ds(c, 1)] = idx + gofs
            qw_ref[p, pl.ds(c, 1)] = w


def _query_prep(qry_xyzT):
    BQ = qry_xyzT.shape[1]
    blk = 2048
    grid = (BQ // blk,)
    return pl.pallas_call(
        _qprep_body,
        grid=grid,
        in_specs=[pl.BlockSpec((3, blk), lambda i: (0, i))],
        out_specs=[
            pl.BlockSpec((3, 4, blk), lambda i: (0, 0, i)),
            pl.BlockSpec((3, 4, blk), lambda i: (0, 0, i)),
        ],
        out_shape=[
            jax.ShapeDtypeStruct((3, 4, BQ), jnp.int32),
            jax.ShapeDtypeStruct((3, 4, BQ), jnp.float32),
        ],
    )(qry_xyzT)


def _splat_body(feat_hbm, cidx_hbm, cw_hbm, planes_hbm,
                shared, zbuf, fbuf, ibuf, wbuf, valbuf, nbuf, obuf):
    cid = lax.axis_index("c")
    sid = lax.axis_index("s")
    N = feat_hbm.shape[1]
    pts_per_tile = N // NS
    nchunk = pts_per_tile // PCH
    rows_per_tile = NROWS // NS
    ZR = zbuf.shape[0]

    def zinit(i, _):
        for t in range(ROWW // 16):
            zbuf[i, pl.ds(16 * t, 16)] = jnp.zeros((16,), jnp.float32)
        return 0
    lax.fori_loop(0, ZR, zinit, 0)

    for kb in range(6):
        b = 2 * cid + (kb // 3)
        p = kb % 3

        for z in range(rows_per_tile // ZR):
            pltpu.sync_copy(
                zbuf, shared.at[pl.ds(sid * rows_per_tile + z * ZR, ZR)])
        plsc.subcore_barrier()

        def chunk_body(j, _):
            pt0 = sid * pts_per_tile + j * PCH
            pltpu.sync_copy(feat_hbm.at[b, pl.ds(pt0, PCH)], fbuf)
            pltpu.sync_copy(cidx_hbm.at[b, p, :, pl.ds(pt0, PCH)], ibuf)
            pltpu.sync_copy(cw_hbm.at[b, p, :, pl.ds(pt0, PCH)], wbuf)

            def q_body(q, _):
                g = q // 16
                l = q - g * 16
                f = [fbuf[q, pl.ds(16 * t, 16)] for t in range(C // 16)]
                for c in range(4):
                    wg = wbuf[c, pl.ds(g * 16, 16)]
                    wv = _lane_bcast(wg, l)
                    row = c * PCH + q
                    for t in range(C // 16):
                        valbuf[row, pl.ds(16 * t, 16)] = f[t] * wv
                    valbuf[row, pl.ds(C, 16)] = wv
                return 0
            lax.fori_loop(0, PCH, q_body, 0)
            for c in range(4):
                pltpu.sync_copy(valbuf.at[pl.ds(c * PCH, PCH)],
                                shared.at[ibuf.at[c]], add=True)
            return 0
        lax.fori_loop(0, nchunk, chunk_body, 0)
        plsc.subcore_barrier()

        NB = nbuf.shape[0]
        def norm_body(r, _):
            row0 = sid * rows_per_tile + r * NB
            pltpu.sync_copy(shared.at[pl.ds(row0, NB)], nbuf)

            def rbody(i, _):
                w = nbuf[i, pl.ds(C, 16)]
                rec = jnp.float32(1.0) / jnp.maximum(w, jnp.float32(1e-6))
                for t in range(C // 16):
                    obuf[i, pl.ds(16 * t, 16)] = nbuf[i, pl.ds(16 * t, 16)] * rec
                return 0
            lax.fori_loop(0, NB, rbody, 0)
            pltpu.sync_copy(obuf, planes_hbm.at[b, p, pl.ds(row0, NB)])
            return 0
        lax.fori_loop(0, rows_per_tile // NB, norm_body, 0)
        plsc.subcore_barrier()


def _splat(ctx_feat, cidx, cw):
    B = ctx_feat.shape[0]
    mesh = plsc.VectorSubcoreMesh(core_axis_name="c", subcore_axis_name="s")
    return pl.kernel(
        _splat_body,
        compiler_params=_SC_PARAMS,
        out_type=jax.ShapeDtypeStruct((B, 3, NROWS, C), jnp.float32),
        mesh=mesh,
        scratch_types=[
            pltpu.VMEM_SHARED((NROWS, ROWW), jnp.float32),
            pltpu.VMEM((32, ROWW), jnp.float32),
            pltpu.VMEM((PCH, C), jnp.float32),
            pltpu.VMEM((4, PCH), jnp.int32),
            pltpu.VMEM((4, PCH), jnp.float32),
            pltpu.VMEM((4 * PCH, ROWW), jnp.float32),
            pltpu.VMEM((64, ROWW), jnp.float32),
            pltpu.VMEM((64, C), jnp.float32),
        ],
    )(ctx_feat, cidx, cw)


def _sample_body(planes_hbm, qidx_hbm, qw_hbm, fused_hbm,
                 ibuf, wbuf, rbuf, obuf, sem):
    cid = lax.axis_index("c")
    sid = lax.axis_index("s")
    wid = sid * NC + cid
    BQ = fused_hbm.shape[0]
    q_per_tile = BQ // (NC * NS)
    nchunk = q_per_tile // QCH

    def chunk_body(j, _):
        qb = wid * q_per_tile + j * QCH
        pltpu.sync_copy(qidx_hbm.at[:, :, pl.ds(qb, QCH)], ibuf)
        pltpu.sync_copy(qw_hbm.at[:, :, pl.ds(qb, QCH)], wbuf)
        cps = []
        for p in range(3):
            for c in range(4):
                cps.append(pltpu.async_copy(
                    planes_hbm.at[ibuf.at[p, c]], rbuf.at[4 * p + c], sem))
        for cp in cps:
            cp.wait()

        def q_body(q, _):
            g = q // 16
            l = q - g * 16
            acc = []
            for p in range(3):
                a = None
                for c in range(4):
                    wg = wbuf[p, c, pl.ds(g * 16, 16)]
                    wv = _lane_bcast(wg, l)
                    t4 = [rbuf[4 * p + c, q, pl.ds(16 * t, 16)] * wv
                          for t in range(C // 16)]
                    a = t4 if a is None else [x + y for x, y in zip(a, t4)]
                acc.append(a)
            for t in range(C // 16):
                obuf[q, pl.ds(16 * t, 16)] = (acc[0][t] * acc[1][t]
                                              * acc[2][t])
            return 0
        lax.fori_loop(0, QCH, q_body, 0)
        pltpu.sync_copy(obuf, fused_hbm.at[pl.ds(qb, QCH)])
        return 0
    lax.fori_loop(0, nchunk, chunk_body, 0)


def _sample(planes_flat, qidx, qw, BQ):
    mesh = plsc.VectorSubcoreMesh(core_axis_name="c", subcore_axis_name="s")
    return pl.kernel(
        _sample_body,
        compiler_params=_SC_PARAMS,
        out_type=jax.ShapeDtypeStruct((BQ, C), jnp.float32),
        mesh=mesh,
        scratch_types=[
            pltpu.VMEM((3, 4, QCH), jnp.int32),
            pltpu.VMEM((3, 4, QCH), jnp.float32),
            pltpu.VMEM((12, QCH, C), jnp.float32),
            pltpu.VMEM((QCH, C), jnp.float32),
            pltpu.SemaphoreType.DMA,
        ],
    )(planes_flat, qidx, qw)


def _qmlp_body(fused_ref, Wq1_ref, bq1_ref, Wq2_ref, bq2_ref, Wd_ref, bd_ref,
               out_ref):
    x = fused_ref[...]
    h1 = _gelu(jnp.dot(x, Wq1_ref[...],
                       preferred_element_type=jnp.float32) + bq1_ref[...])
    h2 = _gelu(jnp.dot(h1, Wq2_ref[...],
                       preferred_element_type=jnp.float32) + bq2_ref[...])
    o = jnp.sum(h2 * Wd_ref[...].reshape(1, H), axis=1, keepdims=True)
    out_ref[...] = o + bd_ref[...]


def _query_mlp(fused, Wq1, bq1, Wq2, bq2, Wd, bd):
    BQ = fused.shape[0]
    blk = 2048
    return pl.pallas_call(
        _qmlp_body,
        grid=(BQ // blk,),
        in_specs=[
            pl.BlockSpec((blk, C), lambda i: (i, 0)),
            pl.BlockSpec((C, H), lambda i: (0, 0)),
            pl.BlockSpec((H,), lambda i: (0,)),
            pl.BlockSpec((H, H), lambda i: (0, 0)),
            pl.BlockSpec((H,), lambda i: (0,)),
            pl.BlockSpec((H, 1), lambda i: (0, 0)),
            pl.BlockSpec((1,), lambda i: (0,)),
        ],
        out_specs=pl.BlockSpec((blk, 1), lambda i: (i, 0)),
        out_shape=jax.ShapeDtypeStruct((BQ, 1), jnp.float32),
    )(fused, Wq1, bq1, Wq2, bq2, Wd, bd)


def kernel(ctx_xyz, ctx_dist, qry_xyz, W1, b1, W2, b2, Wq1, bq1, Wq2, bq2,
           Wd, bd):
    B, N, _ = ctx_xyz.shape
    Q = qry_xyz.shape[1]
    BQ = B * Q

    ctx_feat, cidx, cw = _ctx_mlp(ctx_xyz, ctx_dist, W1, b1, W2, b2)
    planes = _splat(ctx_feat, cidx, cw)
    qidx, qw = _query_prep(qry_xyz.reshape(BQ, 3).T)
    fused = _sample(planes.reshape(B * 3 * NROWS, C), qidx, qw, BQ)
    out = _query_mlp(fused, Wq1, bq1, Wq2, bq2, Wd, bd)
    return out.reshape(B, Q, 1)


# speedup vs baseline: 5.6516x; 1.2364x over previous
import jax
import jax.numpy as jnp
from jax import lax
from jax.experimental import pallas as pl
from jax.experimental.pallas import tpu as pltpu
from jax.experimental.pallas import tpu_sc as plsc

RES = 128
C = 64
H = 256
NROWS = RES * RES
ROWW = C + 16
NC, NS = 2, 16
PCH = 32
QCH = 64

_SC_PARAMS = pltpu.CompilerParams(use_tc_tiling_on_sc=False,
                                  needs_layout_passes=False)


def _gelu(x):
    return 0.5 * x * (1.0 + lax.erf(x * jnp.float32(0.7071067811865476)))


def _to_pixel(c):
    return (jnp.clip(c, -1.0, 1.0) + 1.0) * 0.5 * float(RES - 1)


def _axis_corners(c):
    px = _to_pixel(c)
    i0 = jnp.clip(jnp.floor(px).astype(jnp.int32), 0, RES - 1)
    i1 = jnp.clip(i0 + 1, 0, RES - 1)
    f = jnp.clip(px - i0.astype(jnp.float32), 0.0, 1.0)
    return i0, i1, f


def _plane_corners_rows(xyzT):
    ax = [_axis_corners(xyzT[d:d + 1, :]) for d in range(3)]
    out = []
    for (da, db) in ((0, 1), (0, 2), (1, 2)):
        x0, x1, fx = ax[da]
        y0, y1, fy = ax[db]
        out.append([
            (y0 * RES + x0, (1.0 - fx) * (1.0 - fy)),
            (y0 * RES + x1, fx * (1.0 - fy)),
            (y1 * RES + x0, (1.0 - fx) * fy),
            (y1 * RES + x1, fx * fy),
        ])
    return out


def _lane_bcast(vec16, lane):
    idx = jnp.full((16,), lane, jnp.int32)
    return vec16.at[idx].get(mode="promise_in_bounds")


def _ctx_body(xyz_ref, dist_ref, xyzT_ref, W1_ref, b1_ref, W2_ref, b2_ref,
              feat_ref, cidx_ref, cw_ref):
    blk = xyz_ref.shape[1]
    xyz = xyz_ref[0]
    dist = dist_ref[0]
    x4 = jnp.concatenate([xyz, dist], axis=1)
    h = _gelu(jnp.dot(x4, W1_ref[...],
                      preferred_element_type=jnp.float32) + b1_ref[...])
    feat = jnp.dot(h, W2_ref[...],
                   preferred_element_type=jnp.float32) + b2_ref[...]
    feat_ref[0] = feat

    per_plane = _plane_corners_rows(xyzT_ref[0])
    for p, corners in enumerate(per_plane):
        for c, (idx, w) in enumerate(corners):
            cidx_ref[0, p, pl.ds(c, 1)] = idx
            cw_ref[0, p, pl.ds(c, 1)] = w


def _ctx_mlp(ctx_xyz, ctx_dist, W1, b1, W2, b2):
    B, N, _ = ctx_xyz.shape
    blk = 2048
    grid = (B, N // blk)
    xyzT = ctx_xyz.transpose(0, 2, 1)
    return pl.pallas_call(
        _ctx_body,
        grid=grid,
        in_specs=[
            pl.BlockSpec((1, blk, 3), lambda b, j: (b, j, 0)),
            pl.BlockSpec((1, blk, 1), lambda b, j: (b, j, 0)),
            pl.BlockSpec((1, 3, blk), lambda b, j: (b, 0, j)),
            pl.BlockSpec((4, H), lambda b, j: (0, 0)),
            pl.BlockSpec((H,), lambda b, j: (0,)),
            pl.BlockSpec((H, C), lambda b, j: (0, 0)),
            pl.BlockSpec((C,), lambda b, j: (0,)),
        ],
        out_specs=[
            pl.BlockSpec((1, blk, C), lambda b, j: (b, j, 0)),
            pl.BlockSpec((1, 3, 4, blk), lambda b, j: (b, 0, 0, j)),
            pl.BlockSpec((1, 3, 4, blk), lambda b, j: (b, 0, 0, j)),
        ],
        out_shape=[
            jax.ShapeDtypeStruct((B, N, C), jnp.float32),
            jax.ShapeDtypeStruct((B, 3, 4, N), jnp.int32),
            jax.ShapeDtypeStruct((B, 3, 4, N), jnp.float32),
        ],
    )(ctx_xyz, ctx_dist[..., None], xyzT, W1, b1, W2, b2)


def _qprep_body(xyzT_ref, qidx_ref, qw_ref):
    i = pl.program_id(0)
    nblk_per_batch = pl.num_programs(0) // 4
    b = i // nblk_per_batch
    per_plane = _plane_corners_rows(xyzT_ref[...])
    for p, corners in enumerate(per_plane):
        gofs = (b * 3 + p) * NROWS
        for c, (idx, w) in enumerate(corners):
            qidx_ref[p, pl.ds(c, 1)] = idx + gofs
            qw_ref[p, pl.ds(c, 1)] = w


def _query_prep(qry_xyzT):
    BQ = qry_xyzT.shape[1]
    blk = 2048
    grid = (BQ // blk,)
    return pl.pallas_call(
        _qprep_body,
        grid=grid,
        in_specs=[pl.BlockSpec((3, blk), lambda i: (0, i))],
        out_specs=[
            pl.BlockSpec((3, 4, blk), lambda i: (0, 0, i)),
            pl.BlockSpec((3, 4, blk), lambda i: (0, 0, i)),
        ],
        out_shape=[
            jax.ShapeDtypeStruct((3, 4, BQ), jnp.int32),
            jax.ShapeDtypeStruct((3, 4, BQ), jnp.float32),
        ],
    )(qry_xyzT)


def _splat_body(feat_hbm, cidx_hbm, cw_hbm, planes_hbm,
                shared, zbuf, fbuf, ibuf, wbuf, valbuf, nbuf, obuf,
                lsem0, lsem1, ssem, zsem, nsem):
    cid = lax.axis_index("c")
    sid = lax.axis_index("s")
    N = feat_hbm.shape[1]
    pts_per_tile = N // NS
    nchunk = pts_per_tile // PCH
    rows_per_tile = NROWS // NS
    ZR = zbuf.shape[0]
    lsems = [lsem0, lsem1]

    def zinit(i, _):
        for t in range(ROWW // 16):
            zbuf[i, pl.ds(16 * t, 16)] = jnp.zeros((16,), jnp.float32)
        return 0
    lax.fori_loop(0, ZR, zinit, 0)

    def issue_loads(b, p, j, s):
        pt0 = sid * pts_per_tile + j * PCH
        d1 = pltpu.async_copy(feat_hbm.at[b, pl.ds(pt0, PCH)], fbuf.at[s],
                              lsems[s])
        d2 = pltpu.async_copy(cidx_hbm.at[b, p, :, pl.ds(pt0, PCH)],
                              ibuf.at[s], lsems[s])
        d3 = pltpu.async_copy(cw_hbm.at[b, p, :, pl.ds(pt0, PCH)],
                              wbuf.at[s], lsems[s])
        return d1, d2, d3

    for kb in range(6):
        b = 2 * cid + (kb // 3)
        p = kb % 3

        zds = []
        for z in range(rows_per_tile // ZR):
            zds.append(pltpu.async_copy(
                zbuf, shared.at[pl.ds(sid * rows_per_tile + z * ZR, ZR)],
                zsem))
        for zd in zds:
            zd.wait()
        plsc.subcore_barrier()

        issue_loads(b, p, 0, 0)
        issue_loads(b, p, 1, 1)

        def pipe2(j2, _):
            for s in range(2):
                j = 2 * j2 + s
                pt0 = sid * pts_per_tile + j * PCH
                pltpu.make_async_copy(feat_hbm.at[b, pl.ds(pt0, PCH)],
                                      fbuf.at[s], lsems[s]).wait()
                pltpu.make_async_copy(cidx_hbm.at[b, p, :, pl.ds(pt0, PCH)],
                                      ibuf.at[s], lsems[s]).wait()
                pltpu.make_async_copy(cw_hbm.at[b, p, :, pl.ds(pt0, PCH)],
                                      wbuf.at[s], lsems[s]).wait()

                def q_body(q, _):
                    g = q // 16
                    l = q - g * 16
                    f = [fbuf[s, q, pl.ds(16 * t, 16)]
                         for t in range(C // 16)]
                    for c in range(4):
                        wg = wbuf[s, c, pl.ds(g * 16, 16)]
                        wv = _lane_bcast(wg, l)
                        row = c * PCH + q
                        for t in range(C // 16):
                            valbuf[row, pl.ds(16 * t, 16)] = f[t] * wv
                        valbuf[row, pl.ds(C, 16)] = wv
                    return 0
                lax.fori_loop(0, PCH, q_body, 0)

                jn = lax.rem(j + 2, nchunk)
                issue_loads(b, p, jn, s)

                sds = [pltpu.async_copy(valbuf.at[pl.ds(c * PCH, PCH)],
                                        shared.at[ibuf.at[s, c]], ssem,
                                        add=True)
                       for c in range(4)]
                for sd in sds:
                    sd.wait()
            return 0
        lax.fori_loop(0, nchunk // 2, pipe2, 0)
        for s in range(2):
            pt0 = sid * pts_per_tile + s * PCH
            pltpu.make_async_copy(feat_hbm.at[b, pl.ds(pt0, PCH)],
                                  fbuf.at[s], lsems[s]).wait()
            pltpu.make_async_copy(cidx_hbm.at[b, p, :, pl.ds(pt0, PCH)],
                                  ibuf.at[s], lsems[s]).wait()
            pltpu.make_async_copy(cw_hbm.at[b, p, :, pl.ds(pt0, PCH)],
                                  wbuf.at[s], lsems[s]).wait()
        plsc.subcore_barrier()

        NB = nbuf.shape[0]
        def norm_body(r, _):
            row0 = sid * rows_per_tile + r * NB
            pltpu.async_copy(shared.at[pl.ds(row0, NB)], nbuf, nsem).wait()

            def rbody(i, _):
                w = nbuf[i, pl.ds(C, 16)]
                rec = jnp.float32(1.0) / jnp.maximum(w, jnp.float32(1e-6))
                for t in range(C // 16):
                    obuf[i, pl.ds(16 * t, 16)] = nbuf[i, pl.ds(16 * t, 16)] * rec
                return 0
            lax.fori_loop(0, NB, rbody, 0)
            pltpu.sync_copy(obuf, planes_hbm.at[b, p, pl.ds(row0, NB)])
            return 0
        lax.fori_loop(0, rows_per_tile // NB, norm_body, 0)
        plsc.subcore_barrier()


def _splat(ctx_feat, cidx, cw):
    B = ctx_feat.shape[0]
    mesh = plsc.VectorSubcoreMesh(core_axis_name="c", subcore_axis_name="s")
    return pl.kernel(
        _splat_body,
        compiler_params=_SC_PARAMS,
        out_type=jax.ShapeDtypeStruct((B, 3, NROWS, C), jnp.float32),
        mesh=mesh,
        scratch_types=[
            pltpu.VMEM_SHARED((NROWS, ROWW), jnp.float32),
            pltpu.VMEM((32, ROWW), jnp.float32),
            pltpu.VMEM((2, PCH, C), jnp.float32),
            pltpu.VMEM((2, 4, PCH), jnp.int32),
            pltpu.VMEM((2, 4, PCH), jnp.float32),
            pltpu.VMEM((4 * PCH, ROWW), jnp.float32),
            pltpu.VMEM((128, ROWW), jnp.float32),
            pltpu.VMEM((128, C), jnp.float32),
            pltpu.SemaphoreType.DMA,
            pltpu.SemaphoreType.DMA,
            pltpu.SemaphoreType.DMA,
            pltpu.SemaphoreType.DMA,
            pltpu.SemaphoreType.DMA,
        ],
    )(ctx_feat, cidx, cw)


def _sample_body(planes_hbm, qidx_hbm, qw_hbm, fused_hbm,
                 ibuf, wbuf, rbuf, obuf, sem):
    cid = lax.axis_index("c")
    sid = lax.axis_index("s")
    wid = sid * NC + cid
    BQ = fused_hbm.shape[0]
    q_per_tile = BQ // (NC * NS)
    nchunk = q_per_tile // QCH

    def chunk_body(j, _):
        qb = wid * q_per_tile + j * QCH
        pltpu.sync_copy(qidx_hbm.at[:, :, pl.ds(qb, QCH)], ibuf)
        pltpu.sync_copy(qw_hbm.at[:, :, pl.ds(qb, QCH)], wbuf)
        cps = []
        for p in range(3):
            for c in range(4):
                cps.append(pltpu.async_copy(
                    planes_hbm.at[ibuf.at[p, c]], rbuf.at[4 * p + c], sem))
        for cp in cps:
            cp.wait()

        def q_body(q, _):
            g = q // 16
            l = q - g * 16
            acc = []
            for p in range(3):
                a = None
                for c in range(4):
                    wg = wbuf[p, c, pl.ds(g * 16, 16)]
                    wv = _lane_bcast(wg, l)
                    t4 = [rbuf[4 * p + c, q, pl.ds(16 * t, 16)] * wv
                          for t in range(C // 16)]
                    a = t4 if a is None else [x + y for x, y in zip(a, t4)]
                acc.append(a)
            for t in range(C // 16):
                obuf[q, pl.ds(16 * t, 16)] = (acc[0][t] * acc[1][t]
                                              * acc[2][t])
            return 0
        lax.fori_loop(0, QCH, q_body, 0)
        pltpu.sync_copy(obuf, fused_hbm.at[pl.ds(qb, QCH)])
        return 0
    lax.fori_loop(0, nchunk, chunk_body, 0)


def _sample(planes_flat, qidx, qw, BQ):
    mesh = plsc.VectorSubcoreMesh(core_axis_name="c", subcore_axis_name="s")
    return pl.kernel(
        _sample_body,
        compiler_params=_SC_PARAMS,
        out_type=jax.ShapeDtypeStruct((BQ, C), jnp.float32),
        mesh=mesh,
        scratch_types=[
            pltpu.VMEM((3, 4, QCH), jnp.int32),
            pltpu.VMEM((3, 4, QCH), jnp.float32),
            pltpu.VMEM((12, QCH, C), jnp.float32),
            pltpu.VMEM((QCH, C), jnp.float32),
            pltpu.SemaphoreType.DMA,
        ],
    )(planes_flat, qidx, qw)


def _qmlp_body(fused_ref, Wq1_ref, bq1_ref, Wq2_ref, bq2_ref, Wd_ref, bd_ref,
               out_ref):
    x = fused_ref[...]
    h1 = _gelu(jnp.dot(x, Wq1_ref[...],
                       preferred_element_type=jnp.float32) + bq1_ref[...])
    h2 = _gelu(jnp.dot(h1, Wq2_ref[...],
                       preferred_element_type=jnp.float32) + bq2_ref[...])
    o = jnp.sum(h2 * Wd_ref[...].reshape(1, H), axis=1, keepdims=True)
    out_ref[...] = o + bd_ref[...]


def _query_mlp(fused, Wq1, bq1, Wq2, bq2, Wd, bd):
    BQ = fused.shape[0]
    blk = 2048
    return pl.pallas_call(
        _qmlp_body,
        grid=(BQ // blk,),
        in_specs=[
            pl.BlockSpec((blk, C), lambda i: (i, 0)),
            pl.BlockSpec((C, H), lambda i: (0, 0)),
            pl.BlockSpec((H,), lambda i: (0,)),
            pl.BlockSpec((H, H), lambda i: (0, 0)),
            pl.BlockSpec((H,), lambda i: (0,)),
            pl.BlockSpec((H, 1), lambda i: (0, 0)),
            pl.BlockSpec((1,), lambda i: (0,)),
        ],
        out_specs=pl.BlockSpec((blk, 1), lambda i: (i, 0)),
        out_shape=jax.ShapeDtypeStruct((BQ, 1), jnp.float32),
    )(fused, Wq1, bq1, Wq2, bq2, Wd, bd)


def kernel(ctx_xyz, ctx_dist, qry_xyz, W1, b1, W2, b2, Wq1, bq1, Wq2, bq2,
           Wd, bd):
    B, N, _ = ctx_xyz.shape
    Q = qry_xyz.shape[1]
    BQ = B * Q

    ctx_feat, cidx, cw = _ctx_mlp(ctx_xyz, ctx_dist, W1, b1, W2, b2)
    planes = _splat(ctx_feat, cidx, cw)
    qidx, qw = _query_prep(qry_xyz.reshape(BQ, 3).T)
    fused = _sample(planes.reshape(B * 3 * NROWS, C), qidx, qw, BQ)
    out = _query_mlp(fused, Wq1, bq1, Wq2, bq2, Wd, bd)
    return out.reshape(B, Q, 1)


# speedup vs baseline: 6.3422x; 1.1222x over previous
import jax
import jax.numpy as jnp
from jax import lax
from jax.experimental import pallas as pl
from jax.experimental.pallas import tpu as pltpu
from jax.experimental.pallas import tpu_sc as plsc

RES = 128
C = 64
H = 256
NROWS = RES * RES
ROWW = C + 16
NC, NS = 2, 16
PCH = 32
QCH = 64

_SC_PARAMS = pltpu.CompilerParams(use_tc_tiling_on_sc=False,
                                  needs_layout_passes=False)


def _gelu(x):
    return 0.5 * x * (1.0 + lax.erf(x * jnp.float32(0.7071067811865476)))


def _to_pixel(c):
    return (jnp.clip(c, -1.0, 1.0) + 1.0) * 0.5 * float(RES - 1)


def _axis_corners(c):
    px = _to_pixel(c)
    i0 = jnp.clip(jnp.floor(px).astype(jnp.int32), 0, RES - 1)
    i1 = jnp.clip(i0 + 1, 0, RES - 1)
    f = jnp.clip(px - i0.astype(jnp.float32), 0.0, 1.0)
    return i0, i1, f


def _plane_corners_rows(xyzT):
    ax = [_axis_corners(xyzT[d:d + 1, :]) for d in range(3)]
    out = []
    for (da, db) in ((0, 1), (0, 2), (1, 2)):
        x0, x1, fx = ax[da]
        y0, y1, fy = ax[db]
        out.append([
            (y0 * RES + x0, (1.0 - fx) * (1.0 - fy)),
            (y0 * RES + x1, fx * (1.0 - fy)),
            (y1 * RES + x0, (1.0 - fx) * fy),
            (y1 * RES + x1, fx * fy),
        ])
    return out


def _lane_bcast(vec16, lane):
    idx = jnp.full((16,), lane, jnp.int32)
    return vec16.at[idx].get(mode="promise_in_bounds")


def _ctx_body(xyz_ref, dist_ref, xyzT_ref, W1_ref, b1_ref, W2_ref, b2_ref,
              feat_ref, cidx_ref, cw_ref):
    blk = xyz_ref.shape[1]
    xyz = xyz_ref[0]
    dist = dist_ref[0]
    x4 = jnp.concatenate([xyz, dist], axis=1)
    h = _gelu(jnp.dot(x4, W1_ref[...],
                      preferred_element_type=jnp.float32) + b1_ref[...])
    feat = jnp.dot(h, W2_ref[...],
                   preferred_element_type=jnp.float32) + b2_ref[...]
    feat_ref[0] = feat

    per_plane = _plane_corners_rows(xyzT_ref[0])
    for p, corners in enumerate(per_plane):
        for c, (idx, w) in enumerate(corners):
            cidx_ref[0, p, pl.ds(c, 1)] = idx
            cw_ref[0, p, pl.ds(c, 1)] = w


def _ctx_mlp(ctx_xyz, ctx_dist, W1, b1, W2, b2):
    B, N, _ = ctx_xyz.shape
    blk = 2048
    grid = (B, N // blk)
    xyzT = ctx_xyz.transpose(0, 2, 1)
    return pl.pallas_call(
        _ctx_body,
        grid=grid,
        in_specs=[
            pl.BlockSpec((1, blk, 3), lambda b, j: (b, j, 0)),
            pl.BlockSpec((1, blk, 1), lambda b, j: (b, j, 0)),
            pl.BlockSpec((1, 3, blk), lambda b, j: (b, 0, j)),
            pl.BlockSpec((4, H), lambda b, j: (0, 0)),
            pl.BlockSpec((H,), lambda b, j: (0,)),
            pl.BlockSpec((H, C), lambda b, j: (0, 0)),
            pl.BlockSpec((C,), lambda b, j: (0,)),
        ],
        out_specs=[
            pl.BlockSpec((1, blk, C), lambda b, j: (b, j, 0)),
            pl.BlockSpec((1, 3, 4, blk), lambda b, j: (b, 0, 0, j)),
            pl.BlockSpec((1, 3, 4, blk), lambda b, j: (b, 0, 0, j)),
        ],
        out_shape=[
            jax.ShapeDtypeStruct((B, N, C), jnp.float32),
            jax.ShapeDtypeStruct((B, 3, 4, N), jnp.int32),
            jax.ShapeDtypeStruct((B, 3, 4, N), jnp.float32),
        ],
    )(ctx_xyz, ctx_dist[..., None], xyzT, W1, b1, W2, b2)


def _qprep_body(xyzT_ref, qidx_ref, qw_ref):
    i = pl.program_id(0)
    nblk_per_batch = pl.num_programs(0) // 4
    b = i // nblk_per_batch
    per_plane = _plane_corners_rows(xyzT_ref[...])
    for p, corners in enumerate(per_plane):
        gofs = (b * 3 + p) * NROWS
        for c, (idx, w) in enumerate(corners):
            qidx_ref[p, pl.ds(c, 1)] = idx + gofs
            qw_ref[p, pl.ds(c, 1)] = w


def _query_prep(qry_xyzT):
    BQ = qry_xyzT.shape[1]
    blk = 2048
    grid = (BQ // blk,)
    return pl.pallas_call(
        _qprep_body,
        grid=grid,
        in_specs=[pl.BlockSpec((3, blk), lambda i: (0, i))],
        out_specs=[
            pl.BlockSpec((3, 4, blk), lambda i: (0, 0, i)),
            pl.BlockSpec((3, 4, blk), lambda i: (0, 0, i)),
        ],
        out_shape=[
            jax.ShapeDtypeStruct((3, 4, BQ), jnp.int32),
            jax.ShapeDtypeStruct((3, 4, BQ), jnp.float32),
        ],
    )(qry_xyzT)


def _splat_body(feat_hbm, cidx_hbm, cw_hbm, planes_hbm,
                shared, zbuf, fbuf, ibuf, wbuf, valbuf, nbuf, obuf,
                lsem0, lsem1, ssem, zsem, nsem):
    cid = lax.axis_index("c")
    sid = lax.axis_index("s")
    N = feat_hbm.shape[1]
    pts_per_tile = N // NS
    nchunk = pts_per_tile // PCH
    rows_per_tile = NROWS // NS
    ZR = zbuf.shape[0]
    lsems = [lsem0, lsem1]

    def zinit(i, _):
        for t in range(ROWW // 16):
            zbuf[i, pl.ds(16 * t, 16)] = jnp.zeros((16,), jnp.float32)
        return 0
    lax.fori_loop(0, ZR, zinit, 0)

    def issue_loads(b, p, j, s):
        pt0 = sid * pts_per_tile + j * PCH
        d1 = pltpu.async_copy(feat_hbm.at[b, pl.ds(pt0, PCH)], fbuf.at[s],
                              lsems[s])
        d2 = pltpu.async_copy(cidx_hbm.at[b, p, :, pl.ds(pt0, PCH)],
                              ibuf.at[s], lsems[s])
        d3 = pltpu.async_copy(cw_hbm.at[b, p, :, pl.ds(pt0, PCH)],
                              wbuf.at[s], lsems[s])
        return d1, d2, d3

    for kb in range(6):
        b = 2 * cid + (kb // 3)
        p = kb % 3

        zds = []
        for z in range(rows_per_tile // ZR):
            zds.append(pltpu.async_copy(
                zbuf, shared.at[pl.ds(sid * rows_per_tile + z * ZR, ZR)],
                zsem))
        for zd in zds:
            zd.wait()
        plsc.subcore_barrier()

        issue_loads(b, p, 0, 0)
        issue_loads(b, p, 1, 1)

        def pipe2(j2, _):
            for s in range(2):
                j = 2 * j2 + s
                pt0 = sid * pts_per_tile + j * PCH
                pltpu.make_async_copy(feat_hbm.at[b, pl.ds(pt0, PCH)],
                                      fbuf.at[s], lsems[s]).wait()
                pltpu.make_async_copy(cidx_hbm.at[b, p, :, pl.ds(pt0, PCH)],
                                      ibuf.at[s], lsems[s]).wait()
                pltpu.make_async_copy(cw_hbm.at[b, p, :, pl.ds(pt0, PCH)],
                                      wbuf.at[s], lsems[s]).wait()

                def q_body(q, _):
                    g = q // 16
                    l = q - g * 16
                    f = [fbuf[s, q, pl.ds(16 * t, 16)]
                         for t in range(C // 16)]
                    for c in range(4):
                        wg = wbuf[s, c, pl.ds(g * 16, 16)]
                        wv = _lane_bcast(wg, l)
                        row = c * PCH + q
                        for t in range(C // 16):
                            valbuf[row, pl.ds(16 * t, 16)] = f[t] * wv
                        valbuf[row, pl.ds(C, 16)] = wv
                    return 0
                lax.fori_loop(0, PCH, q_body, 0)

                jn = lax.rem(j + 2, nchunk)
                issue_loads(b, p, jn, s)

                sds = [pltpu.async_copy(valbuf.at[pl.ds(c * PCH, PCH)],
                                        shared.at[ibuf.at[s, c]], ssem,
                                        add=True)
                       for c in range(4)]
                for sd in sds:
                    sd.wait()
            return 0
        lax.fori_loop(0, nchunk // 2, pipe2, 0)
        for s in range(2):
            pt0 = sid * pts_per_tile + s * PCH
            pltpu.make_async_copy(feat_hbm.at[b, pl.ds(pt0, PCH)],
                                  fbuf.at[s], lsems[s]).wait()
            pltpu.make_async_copy(cidx_hbm.at[b, p, :, pl.ds(pt0, PCH)],
                                  ibuf.at[s], lsems[s]).wait()
            pltpu.make_async_copy(cw_hbm.at[b, p, :, pl.ds(pt0, PCH)],
                                  wbuf.at[s], lsems[s]).wait()
        plsc.subcore_barrier()

        NB = nbuf.shape[0]
        def norm_body(r, _):
            row0 = sid * rows_per_tile + r * NB
            pltpu.async_copy(shared.at[pl.ds(row0, NB)], nbuf, nsem).wait()

            def rbody(i, _):
                w = nbuf[i, pl.ds(C, 16)]
                rec = jnp.float32(1.0) / jnp.maximum(w, jnp.float32(1e-6))
                for t in range(C // 16):
                    obuf[i, pl.ds(16 * t, 16)] = nbuf[i, pl.ds(16 * t, 16)] * rec
                return 0
            lax.fori_loop(0, NB, rbody, 0)
            pltpu.sync_copy(obuf, planes_hbm.at[b, p, pl.ds(row0, NB)])
            return 0
        lax.fori_loop(0, rows_per_tile // NB, norm_body, 0)
        plsc.subcore_barrier()


def _splat(ctx_feat, cidx, cw):
    B = ctx_feat.shape[0]
    mesh = plsc.VectorSubcoreMesh(core_axis_name="c", subcore_axis_name="s")
    return pl.kernel(
        _splat_body,
        compiler_params=_SC_PARAMS,
        out_type=jax.ShapeDtypeStruct((B, 3, NROWS, C), jnp.float32),
        mesh=mesh,
        scratch_types=[
            pltpu.VMEM_SHARED((NROWS, ROWW), jnp.float32),
            pltpu.VMEM((32, ROWW), jnp.float32),
            pltpu.VMEM((2, PCH, C), jnp.float32),
            pltpu.VMEM((2, 4, PCH), jnp.int32),
            pltpu.VMEM((2, 4, PCH), jnp.float32),
            pltpu.VMEM((4 * PCH, ROWW), jnp.float32),
            pltpu.VMEM((128, ROWW), jnp.float32),
            pltpu.VMEM((128, C), jnp.float32),
            pltpu.SemaphoreType.DMA,
            pltpu.SemaphoreType.DMA,
            pltpu.SemaphoreType.DMA,
            pltpu.SemaphoreType.DMA,
            pltpu.SemaphoreType.DMA,
        ],
    )(ctx_feat, cidx, cw)


def _sample_body(planes_hbm, qidx_hbm, qw_hbm, fused_hbm,
                 ibuf, wbuf, rbuf, obuf, gsem0, gsem1):
    cid = lax.axis_index("c")
    sid = lax.axis_index("s")
    wid = sid * NC + cid
    BQ = fused_hbm.shape[0]
    q_per_tile = BQ // (NC * NS)
    nchunk = q_per_tile // QCH
    gsems = [gsem0, gsem1]

    def stage(j, s):
        qb = wid * q_per_tile + j * QCH
        pltpu.sync_copy(qidx_hbm.at[:, :, pl.ds(qb, QCH)], ibuf.at[s])
        pltpu.sync_copy(qw_hbm.at[:, :, pl.ds(qb, QCH)], wbuf.at[s])
        for p in range(3):
            for c in range(4):
                pltpu.async_copy(planes_hbm.at[ibuf.at[s, p, c]],
                                 rbuf.at[s, 4 * p + c], gsems[s])

    def drain(s):
        for p in range(3):
            for c in range(4):
                pltpu.make_async_copy(planes_hbm.at[ibuf.at[s, p, c]],
                                      rbuf.at[s, 4 * p + c],
                                      gsems[s]).wait()

    stage(0, 0)
    stage(1, 1)

    def pipe2(j2, _):
        for s in range(2):
            j = 2 * j2 + s
            drain(s)

            def q_body(q, _):
                g = q // 16
                l = q - g * 16
                acc = []
                for p in range(3):
                    a = None
                    for c in range(4):
                        wg = wbuf[s, p, c, pl.ds(g * 16, 16)]
                        wv = _lane_bcast(wg, l)
                        t4 = [rbuf[s, 4 * p + c, q, pl.ds(16 * t, 16)] * wv
                              for t in range(C // 16)]
                        a = t4 if a is None else [x + y
                                                  for x, y in zip(a, t4)]
                    acc.append(a)
                for t in range(C // 16):
                    obuf[q, pl.ds(16 * t, 16)] = (acc[0][t] * acc[1][t]
                                                  * acc[2][t])
                return 0
            lax.fori_loop(0, QCH, q_body, 0)
            qb = wid * q_per_tile + j * QCH
            pltpu.sync_copy(obuf, fused_hbm.at[pl.ds(qb, QCH)])
            stage(lax.rem(j + 2, nchunk), s)
        return 0
    lax.fori_loop(0, nchunk // 2, pipe2, 0)
    drain(0)
    drain(1)


def _sample(planes_flat, qidx, qw, BQ):
    mesh = plsc.VectorSubcoreMesh(core_axis_name="c", subcore_axis_name="s")
    return pl.kernel(
        _sample_body,
        compiler_params=_SC_PARAMS,
        out_type=jax.ShapeDtypeStruct((BQ, C), jnp.float32),
        mesh=mesh,
        scratch_types=[
            pltpu.VMEM((2, 3, 4, QCH), jnp.int32),
            pltpu.VMEM((2, 3, 4, QCH), jnp.float32),
            pltpu.VMEM((2, 12, QCH, C), jnp.float32),
            pltpu.VMEM((QCH, C), jnp.float32),
            pltpu.SemaphoreType.DMA,
            pltpu.SemaphoreType.DMA,
        ],
    )(planes_flat, qidx, qw)


def _qmlp_body(fused_ref, Wq1_ref, bq1_ref, Wq2_ref, bq2_ref, Wd_ref, bd_ref,
               out_ref):
    x = fused_ref[...]
    h1 = _gelu(jnp.dot(x, Wq1_ref[...],
                       preferred_element_type=jnp.float32) + bq1_ref[...])
    h2 = _gelu(jnp.dot(h1, Wq2_ref[...],
                       preferred_element_type=jnp.float32) + bq2_ref[...])
    o = jnp.sum(h2 * Wd_ref[...].reshape(1, H), axis=1, keepdims=True)
    out_ref[...] = o + bd_ref[...]


def _query_mlp(fused, Wq1, bq1, Wq2, bq2, Wd, bd):
    BQ = fused.shape[0]
    blk = 4096
    return pl.pallas_call(
        _qmlp_body,
        grid=(BQ // blk,),
        in_specs=[
            pl.BlockSpec((blk, C), lambda i: (i, 0)),
            pl.BlockSpec((C, H), lambda i: (0, 0)),
            pl.BlockSpec((H,), lambda i: (0,)),
            pl.BlockSpec((H, H), lambda i: (0, 0)),
            pl.BlockSpec((H,), lambda i: (0,)),
            pl.BlockSpec((H, 1), lambda i: (0, 0)),
            pl.BlockSpec((1,), lambda i: (0,)),
        ],
        out_specs=pl.BlockSpec((blk, 1), lambda i: (i, 0)),
        out_shape=jax.ShapeDtypeStruct((BQ, 1), jnp.float32),
    )(fused, Wq1, bq1, Wq2, bq2, Wd, bd)


def kernel(ctx_xyz, ctx_dist, qry_xyz, W1, b1, W2, b2, Wq1, bq1, Wq2, bq2,
           Wd, bd):
    B, N, _ = ctx_xyz.shape
    Q = qry_xyz.shape[1]
    BQ = B * Q

    ctx_feat, cidx, cw = _ctx_mlp(ctx_xyz, ctx_dist, W1, b1, W2, b2)
    planes = _splat(ctx_feat, cidx, cw)
    qidx, qw = _query_prep(qry_xyz.reshape(BQ, 3).T)
    fused = _sample(planes.reshape(B * 3 * NROWS, C), qidx, qw, BQ)
    out = _query_mlp(fused, Wq1, bq1, Wq2, bq2, Wd, bd)
    return out.reshape(B, Q, 1)


# speedup vs baseline: 6.7226x; 1.0600x over previous
import jax
import jax.numpy as jnp
from jax import lax
from jax.experimental import pallas as pl
from jax.experimental.pallas import tpu as pltpu
from jax.experimental.pallas import tpu_sc as plsc

RES = 128
C = 64
H = 256
NROWS = RES * RES
ROWW = C + 16
NC, NS = 2, 16
PCH = 32
QCH = 64

_SC_PARAMS = pltpu.CompilerParams(use_tc_tiling_on_sc=False,
                                  needs_layout_passes=False)


def _gelu(x):
    return 0.5 * x * (1.0 + lax.erf(x * jnp.float32(0.7071067811865476)))


def _to_pixel(c):
    return (jnp.clip(c, -1.0, 1.0) + 1.0) * 0.5 * float(RES - 1)


def _axis_corners(c):
    px = _to_pixel(c)
    i0 = jnp.clip(jnp.floor(px).astype(jnp.int32), 0, RES - 1)
    i1 = jnp.clip(i0 + 1, 0, RES - 1)
    f = jnp.clip(px - i0.astype(jnp.float32), 0.0, 1.0)
    return i0, i1, f


def _plane_corners_rows(xyzT):
    ax = [_axis_corners(xyzT[d:d + 1, :]) for d in range(3)]
    out = []
    for (da, db) in ((0, 1), (0, 2), (1, 2)):
        x0, x1, fx = ax[da]
        y0, y1, fy = ax[db]
        out.append([
            (y0 * RES + x0, (1.0 - fx) * (1.0 - fy)),
            (y0 * RES + x1, fx * (1.0 - fy)),
            (y1 * RES + x0, (1.0 - fx) * fy),
            (y1 * RES + x1, fx * fy),
        ])
    return out


def _lane_bcast(vec16, lane):
    idx = jnp.full((16,), lane, jnp.int32)
    return vec16.at[idx].get(mode="promise_in_bounds")


def _ctx_body(xyz_ref, dist_ref, xyzT_ref, W1_ref, b1_ref, W2_ref, b2_ref,
              feat_ref, cidx_ref, cw_ref):
    blk = xyz_ref.shape[1]
    xyz = xyz_ref[0]
    dist = dist_ref[0]
    x4 = jnp.concatenate([xyz, dist], axis=1)
    h = _gelu(jnp.dot(x4, W1_ref[...],
                      preferred_element_type=jnp.float32) + b1_ref[...])
    feat = jnp.dot(h, W2_ref[...],
                   preferred_element_type=jnp.float32) + b2_ref[...]
    feat_ref[0] = feat

    per_plane = _plane_corners_rows(xyzT_ref[0])
    for p, corners in enumerate(per_plane):
        for c, (idx, w) in enumerate(corners):
            cidx_ref[0, p, pl.ds(c, 1)] = idx
            cw_ref[0, p, pl.ds(c, 1)] = w


def _ctx_mlp(ctx_xyz, ctx_dist, W1, b1, W2, b2):
    B, N, _ = ctx_xyz.shape
    blk = 2048
    grid = (B, N // blk)
    xyzT = ctx_xyz.transpose(0, 2, 1)
    return pl.pallas_call(
        _ctx_body,
        grid=grid,
        in_specs=[
            pl.BlockSpec((1, blk, 3), lambda b, j: (b, j, 0)),
            pl.BlockSpec((1, blk, 1), lambda b, j: (b, j, 0)),
            pl.BlockSpec((1, 3, blk), lambda b, j: (b, 0, j)),
            pl.BlockSpec((4, H), lambda b, j: (0, 0)),
            pl.BlockSpec((H,), lambda b, j: (0,)),
            pl.BlockSpec((H, C), lambda b, j: (0, 0)),
            pl.BlockSpec((C,), lambda b, j: (0,)),
        ],
        out_specs=[
            pl.BlockSpec((1, blk, C), lambda b, j: (b, j, 0)),
            pl.BlockSpec((1, 3, 4, blk), lambda b, j: (b, 0, 0, j)),
            pl.BlockSpec((1, 3, 4, blk), lambda b, j: (b, 0, 0, j)),
        ],
        out_shape=[
            jax.ShapeDtypeStruct((B, N, C), jnp.float32),
            jax.ShapeDtypeStruct((B, 3, 4, N), jnp.int32),
            jax.ShapeDtypeStruct((B, 3, 4, N), jnp.float32),
        ],
    )(ctx_xyz, ctx_dist[..., None], xyzT, W1, b1, W2, b2)


def _qprep_body(xyzT_ref, qidx_ref, qw_ref):
    i = pl.program_id(0)
    nblk_per_batch = pl.num_programs(0) // 4
    b = i // nblk_per_batch
    per_plane = _plane_corners_rows(xyzT_ref[...])
    for p, corners in enumerate(per_plane):
        gofs = (b * 3 + p) * NROWS
        for c, (idx, w) in enumerate(corners):
            qidx_ref[p, pl.ds(c, 1)] = idx + gofs
            qw_ref[p, pl.ds(c, 1)] = w


def _query_prep(qry_xyzT):
    BQ = qry_xyzT.shape[1]
    blk = 2048
    grid = (BQ // blk,)
    return pl.pallas_call(
        _qprep_body,
        grid=grid,
        in_specs=[pl.BlockSpec((3, blk), lambda i: (0, i))],
        out_specs=[
            pl.BlockSpec((3, 4, blk), lambda i: (0, 0, i)),
            pl.BlockSpec((3, 4, blk), lambda i: (0, 0, i)),
        ],
        out_shape=[
            jax.ShapeDtypeStruct((3, 4, BQ), jnp.int32),
            jax.ShapeDtypeStruct((3, 4, BQ), jnp.float32),
        ],
    )(qry_xyzT)


def _splat_body(feat_hbm, cidx_hbm, cw_hbm, planes_hbm,
                shared, zbuf, fbuf, ibuf, wbuf, valbuf, nbuf, obuf,
                lsem0, lsem1, ssem, zsem, nsem):
    cid = lax.axis_index("c")
    sid = lax.axis_index("s")
    N = feat_hbm.shape[1]
    pts_per_tile = N // NS
    nchunk = pts_per_tile // PCH
    rows_per_tile = NROWS // NS
    ZR = zbuf.shape[0]
    lsems = [lsem0, lsem1]

    def zinit(i, _):
        for t in range(ROWW // 16):
            zbuf[i, pl.ds(16 * t, 16)] = jnp.zeros((16,), jnp.float32)
        return 0
    lax.fori_loop(0, ZR, zinit, 0)

    def issue_loads(b, p, j, s):
        pt0 = sid * pts_per_tile + j * PCH
        pltpu.async_copy(feat_hbm.at[b, pl.ds(pt0, PCH)], fbuf.at[s],
                         lsems[s])
        pltpu.async_copy(cidx_hbm.at[b, p, :, pl.ds(pt0, PCH)],
                         ibuf.at[lax.rem(j, 4)], lsems[s])
        pltpu.async_copy(cw_hbm.at[b, p, :, pl.ds(pt0, PCH)],
                         wbuf.at[s], lsems[s])

    for kb in range(6):
        b = 2 * cid + (kb // 3)
        p = kb % 3

        zds = []
        for z in range(rows_per_tile // ZR):
            zds.append(pltpu.async_copy(
                zbuf, shared.at[pl.ds(sid * rows_per_tile + z * ZR, ZR)],
                zsem))
        for zd in zds:
            zd.wait()
        plsc.subcore_barrier()

        issue_loads(b, p, 0, 0)
        issue_loads(b, p, 1, 1)

        def drain_scatters(s):
            for c in range(4):
                pltpu.make_async_copy(valbuf.at[s, pl.ds(c * PCH, PCH)],
                                      shared.at[ibuf.at[s, c]],
                                      ssem).wait()

        def drain_loads(j, s):
            pt0 = sid * pts_per_tile + j * PCH
            pltpu.make_async_copy(feat_hbm.at[b, pl.ds(pt0, PCH)],
                                  fbuf.at[s], lsems[s]).wait()
            pltpu.make_async_copy(cidx_hbm.at[b, p, :, pl.ds(pt0, PCH)],
                                  ibuf.at[lax.rem(j, 4)], lsems[s]).wait()
            pltpu.make_async_copy(cw_hbm.at[b, p, :, pl.ds(pt0, PCH)],
                                  wbuf.at[s], lsems[s]).wait()

        def pipe2(j2, _):
            for s in range(2):
                j = 2 * j2 + s
                jm4 = lax.rem(j, 4)
                drain_loads(j, s)

                @pl.when(j2 > 0)
                def _():
                    drain_scatters(s)

                def q_body(q, _):
                    g = q // 16
                    l = q - g * 16
                    f = [fbuf[s, q, pl.ds(16 * t, 16)]
                         for t in range(C // 16)]
                    for c in range(4):
                        wg = wbuf[s, c, pl.ds(g * 16, 16)]
                        wv = _lane_bcast(wg, l)
                        row = c * PCH + q
                        for t in range(C // 16):
                            valbuf[s, row, pl.ds(16 * t, 16)] = f[t] * wv
                        valbuf[s, row, pl.ds(C, 16)] = wv
                    return 0
                lax.fori_loop(0, PCH, q_body, 0, unroll=2)

                for c in range(4):
                    pltpu.async_copy(valbuf.at[s, pl.ds(c * PCH, PCH)],
                                     shared.at[ibuf.at[jm4, c]], ssem,
                                     add=True)

                issue_loads(b, p, lax.rem(j + 2, nchunk), s)
            return 0
        lax.fori_loop(0, nchunk // 2, pipe2, 0)
        drain_scatters(0)
        drain_scatters(1)
        drain_loads(0, 0)
        drain_loads(1, 1)
        plsc.subcore_barrier()

        NB = nbuf.shape[0]
        def norm_body(r, _):
            row0 = sid * rows_per_tile + r * NB
            pltpu.async_copy(shared.at[pl.ds(row0, NB)], nbuf, nsem).wait()

            def rbody(i, _):
                w = nbuf[i, pl.ds(C, 16)]
                rec = jnp.float32(1.0) / jnp.maximum(w, jnp.float32(1e-6))
                for t in range(C // 16):
                    obuf[i, pl.ds(16 * t, 16)] = nbuf[i, pl.ds(16 * t, 16)] * rec
                return 0
            lax.fori_loop(0, NB, rbody, 0)
            pltpu.sync_copy(obuf, planes_hbm.at[b, p, pl.ds(row0, NB)])
            return 0
        lax.fori_loop(0, rows_per_tile // NB, norm_body, 0)
        plsc.subcore_barrier()


def _splat(ctx_feat, cidx, cw):
    B = ctx_feat.shape[0]
    mesh = plsc.VectorSubcoreMesh(core_axis_name="c", subcore_axis_name="s")
    return pl.kernel(
        _splat_body,
        compiler_params=_SC_PARAMS,
        out_type=jax.ShapeDtypeStruct((B, 3, NROWS, C), jnp.float32),
        mesh=mesh,
        scratch_types=[
            pltpu.VMEM_SHARED((NROWS, ROWW), jnp.float32),
            pltpu.VMEM((32, ROWW), jnp.float32),
            pltpu.VMEM((2, PCH, C), jnp.float32),
            pltpu.VMEM((4, 4, PCH), jnp.int32),
            pltpu.VMEM((2, 4, PCH), jnp.float32),
            pltpu.VMEM((2, 4 * PCH, ROWW), jnp.float32),
            pltpu.VMEM((128, ROWW), jnp.float32),
            pltpu.VMEM((128, C), jnp.float32),
            pltpu.SemaphoreType.DMA,
            pltpu.SemaphoreType.DMA,
            pltpu.SemaphoreType.DMA,
            pltpu.SemaphoreType.DMA,
            pltpu.SemaphoreType.DMA,
        ],
    )(ctx_feat, cidx, cw)


def _sample_body(planes_hbm, qidx_hbm, qw_hbm, fused_hbm,
                 ibuf, wbuf, rbuf, obuf, gsem0, gsem1):
    cid = lax.axis_index("c")
    sid = lax.axis_index("s")
    wid = sid * NC + cid
    BQ = fused_hbm.shape[0]
    q_per_tile = BQ // (NC * NS)
    nchunk = q_per_tile // QCH
    gsems = [gsem0, gsem1]

    def stage(j, s):
        qb = wid * q_per_tile + j * QCH
        pltpu.sync_copy(qidx_hbm.at[:, :, pl.ds(qb, QCH)], ibuf.at[s])
        pltpu.sync_copy(qw_hbm.at[:, :, pl.ds(qb, QCH)], wbuf.at[s])
        for p in range(3):
            for c in range(4):
                pltpu.async_copy(planes_hbm.at[ibuf.at[s, p, c]],
                                 rbuf.at[s, 4 * p + c], gsems[s])

    def drain(s):
        for p in range(3):
            for c in range(4):
                pltpu.make_async_copy(planes_hbm.at[ibuf.at[s, p, c]],
                                      rbuf.at[s, 4 * p + c],
                                      gsems[s]).wait()

    stage(0, 0)
    stage(1, 1)

    def pipe2(j2, _):
        for s in range(2):
            j = 2 * j2 + s
            drain(s)

            def q_body(q, _):
                g = q // 16
                l = q - g * 16
                acc = []
                for p in range(3):
                    a = None
                    for c in range(4):
                        wg = wbuf[s, p, c, pl.ds(g * 16, 16)]
                        wv = _lane_bcast(wg, l)
                        t4 = [rbuf[s, 4 * p + c, q, pl.ds(16 * t, 16)] * wv
                              for t in range(C // 16)]
                        a = t4 if a is None else [x + y
                                                  for x, y in zip(a, t4)]
                    acc.append(a)
                for t in range(C // 16):
                    obuf[q, pl.ds(16 * t, 16)] = (acc[0][t] * acc[1][t]
                                                  * acc[2][t])
                return 0
            lax.fori_loop(0, QCH, q_body, 0)
            qb = wid * q_per_tile + j * QCH
            pltpu.sync_copy(obuf, fused_hbm.at[pl.ds(qb, QCH)])
            stage(lax.rem(j + 2, nchunk), s)
        return 0
    lax.fori_loop(0, nchunk // 2, pipe2, 0)
    drain(0)
    drain(1)


def _sample(planes_flat, qidx, qw, BQ):
    mesh = plsc.VectorSubcoreMesh(core_axis_name="c", subcore_axis_name="s")
    return pl.kernel(
        _sample_body,
        compiler_params=_SC_PARAMS,
        out_type=jax.ShapeDtypeStruct((BQ, C), jnp.float32),
        mesh=mesh,
        scratch_types=[
            pltpu.VMEM((2, 3, 4, QCH), jnp.int32),
            pltpu.VMEM((2, 3, 4, QCH), jnp.float32),
            pltpu.VMEM((2, 12, QCH, C), jnp.float32),
            pltpu.VMEM((QCH, C), jnp.float32),
            pltpu.SemaphoreType.DMA,
            pltpu.SemaphoreType.DMA,
        ],
    )(planes_flat, qidx, qw)


def _qmlp_body(fused_ref, Wq1_ref, bq1_ref, Wq2_ref, bq2_ref, Wd_ref, bd_ref,
               out_ref):
    x = fused_ref[...]
    h1 = _gelu(jnp.dot(x, Wq1_ref[...],
                       preferred_element_type=jnp.float32) + bq1_ref[...])
    h2 = _gelu(jnp.dot(h1, Wq2_ref[...],
                       preferred_element_type=jnp.float32) + bq2_ref[...])
    o = jnp.sum(h2 * Wd_ref[...].reshape(1, H), axis=1, keepdims=True)
    out_ref[...] = o + bd_ref[...]


def _query_mlp(fused, Wq1, bq1, Wq2, bq2, Wd, bd):
    BQ = fused.shape[0]
    blk = 4096
    return pl.pallas_call(
        _qmlp_body,
        grid=(BQ // blk,),
        in_specs=[
            pl.BlockSpec((blk, C), lambda i: (i, 0)),
            pl.BlockSpec((C, H), lambda i: (0, 0)),
            pl.BlockSpec((H,), lambda i: (0,)),
            pl.BlockSpec((H, H), lambda i: (0, 0)),
            pl.BlockSpec((H,), lambda i: (0,)),
            pl.BlockSpec((H, 1), lambda i: (0, 0)),
            pl.BlockSpec((1,), lambda i: (0,)),
        ],
        out_specs=pl.BlockSpec((blk, 1), lambda i: (i, 0)),
        out_shape=jax.ShapeDtypeStruct((BQ, 1), jnp.float32),
    )(fused, Wq1, bq1, Wq2, bq2, Wd, bd)


def kernel(ctx_xyz, ctx_dist, qry_xyz, W1, b1, W2, b2, Wq1, bq1, Wq2, bq2,
           Wd, bd):
    B, N, _ = ctx_xyz.shape
    Q = qry_xyz.shape[1]
    BQ = B * Q

    ctx_feat, cidx, cw = _ctx_mlp(ctx_xyz, ctx_dist, W1, b1, W2, b2)
    planes = _splat(ctx_feat, cidx, cw)
    qidx, qw = _query_prep(qry_xyz.reshape(BQ, 3).T)
    fused = _sample(planes.reshape(B * 3 * NROWS, C), qidx, qw, BQ)
    out = _query_mlp(fused, Wq1, bq1, Wq2, bq2, Wd, bd)
    return out.reshape(B, Q, 1)


# speedup vs baseline: 7.1626x; 1.0655x over previous
import jax
import jax.numpy as jnp
from jax import lax
from jax.experimental import pallas as pl
from jax.experimental.pallas import tpu as pltpu
from jax.experimental.pallas import tpu_sc as plsc

RES = 128
C = 64
H = 256
NROWS = RES * RES
ROWW = C + 16
NC, NS = 2, 16
PCH = 32
QCH = 64

_SC_PARAMS = pltpu.CompilerParams(use_tc_tiling_on_sc=False,
                                  needs_layout_passes=False)


def _gelu(x):
    return 0.5 * x * (1.0 + lax.erf(x * jnp.float32(0.7071067811865476)))


def _to_pixel(c):
    return (jnp.clip(c, -1.0, 1.0) + 1.0) * 0.5 * float(RES - 1)


def _axis_corners(c):
    px = _to_pixel(c)
    i0 = jnp.clip(jnp.floor(px).astype(jnp.int32), 0, RES - 1)
    i1 = jnp.clip(i0 + 1, 0, RES - 1)
    f = jnp.clip(px - i0.astype(jnp.float32), 0.0, 1.0)
    return i0, i1, f


def _plane_corners_rows(xyzT):
    ax = [_axis_corners(xyzT[d:d + 1, :]) for d in range(3)]
    out = []
    for (da, db) in ((0, 1), (0, 2), (1, 2)):
        x0, x1, fx = ax[da]
        y0, y1, fy = ax[db]
        out.append([
            (y0 * RES + x0, (1.0 - fx) * (1.0 - fy)),
            (y0 * RES + x1, fx * (1.0 - fy)),
            (y1 * RES + x0, (1.0 - fx) * fy),
            (y1 * RES + x1, fx * fy),
        ])
    return out


def _lane_bcast(vec16, lane):
    idx = jnp.full((16,), lane, jnp.int32)
    return vec16.at[idx].get(mode="promise_in_bounds")


def _ctx_body(xyz_ref, dist_ref, xyzT_ref, W1_ref, b1_ref, W2_ref, b2_ref,
              feat_ref, cidx_ref, cw_ref):
    blk = xyz_ref.shape[1]
    xyz = xyz_ref[0]
    dist = dist_ref[0]
    x4 = jnp.concatenate([xyz, dist], axis=1)
    h = _gelu(jnp.dot(x4, W1_ref[...],
                      preferred_element_type=jnp.float32) + b1_ref[...])
    feat = jnp.dot(h, W2_ref[...],
                   preferred_element_type=jnp.float32) + b2_ref[...]
    feat_ref[0] = feat

    per_plane = _plane_corners_rows(xyzT_ref[0])
    for p, corners in enumerate(per_plane):
        for c, (idx, w) in enumerate(corners):
            cidx_ref[0, p, pl.ds(c, 1)] = idx
            cw_ref[0, p, pl.ds(c, 1)] = w


def _ctx_mlp(ctx_xyz, ctx_dist, W1, b1, W2, b2):
    B, N, _ = ctx_xyz.shape
    blk = 2048
    grid = (B, N // blk)
    xyzT = ctx_xyz.transpose(0, 2, 1)
    return pl.pallas_call(
        _ctx_body,
        grid=grid,
        in_specs=[
            pl.BlockSpec((1, blk, 3), lambda b, j: (b, j, 0)),
            pl.BlockSpec((1, blk, 1), lambda b, j: (b, j, 0)),
            pl.BlockSpec((1, 3, blk), lambda b, j: (b, 0, j)),
            pl.BlockSpec((4, H), lambda b, j: (0, 0)),
            pl.BlockSpec((H,), lambda b, j: (0,)),
            pl.BlockSpec((H, C), lambda b, j: (0, 0)),
            pl.BlockSpec((C,), lambda b, j: (0,)),
        ],
        out_specs=[
            pl.BlockSpec((1, blk, C), lambda b, j: (b, j, 0)),
            pl.BlockSpec((1, 3, 4, blk), lambda b, j: (b, 0, 0, j)),
            pl.BlockSpec((1, 3, 4, blk), lambda b, j: (b, 0, 0, j)),
        ],
        out_shape=[
            jax.ShapeDtypeStruct((B, N, C), jnp.float32),
            jax.ShapeDtypeStruct((B, 3, 4, N), jnp.int32),
            jax.ShapeDtypeStruct((B, 3, 4, N), jnp.float32),
        ],
    )(ctx_xyz, ctx_dist[..., None], xyzT, W1, b1, W2, b2)


def _qprep_body(xyzT_ref, qidx_ref, qw_ref):
    i = pl.program_id(0)
    nblk_per_batch = pl.num_programs(0) // 4
    b = i // nblk_per_batch
    per_plane = _plane_corners_rows(xyzT_ref[...])
    for p, corners in enumerate(per_plane):
        gofs = (b * 3 + p) * NROWS
        for c, (idx, w) in enumerate(corners):
            qidx_ref[p, pl.ds(c, 1)] = idx + gofs
            qw_ref[p, pl.ds(c, 1)] = w


def _query_prep(qry_xyzT):
    BQ = qry_xyzT.shape[1]
    blk = 2048
    grid = (BQ // blk,)
    return pl.pallas_call(
        _qprep_body,
        grid=grid,
        in_specs=[pl.BlockSpec((3, blk), lambda i: (0, i))],
        out_specs=[
            pl.BlockSpec((3, 4, blk), lambda i: (0, 0, i)),
            pl.BlockSpec((3, 4, blk), lambda i: (0, 0, i)),
        ],
        out_shape=[
            jax.ShapeDtypeStruct((3, 4, BQ), jnp.int32),
            jax.ShapeDtypeStruct((3, 4, BQ), jnp.float32),
        ],
    )(qry_xyzT)


def _splat_body(feat_hbm, cidx_hbm, cw_hbm, planes_hbm,
                shared, zbuf, fbuf, ibuf, wbuf, valbuf, nbuf, obuf,
                lsem0, lsem1, ssem, zsem, nsem):
    cid = lax.axis_index("c")
    sid = lax.axis_index("s")
    N = feat_hbm.shape[1]
    pts_per_tile = N // NS
    nchunk = pts_per_tile // PCH
    rows_per_tile = NROWS // NS
    ZR = zbuf.shape[0]
    lsems = [lsem0, lsem1]

    def zinit(i, _):
        for t in range(ROWW // 16):
            zbuf[i, pl.ds(16 * t, 16)] = jnp.zeros((16,), jnp.float32)
        return 0
    lax.fori_loop(0, ZR, zinit, 0)

    def issue_loads(b, p, j, s):
        pt0 = sid * pts_per_tile + j * PCH
        pltpu.async_copy(feat_hbm.at[b, pl.ds(pt0, PCH)], fbuf.at[s],
                         lsems[s])
        pltpu.async_copy(cidx_hbm.at[b, p, :, pl.ds(pt0, PCH)],
                         ibuf.at[lax.rem(j, 4)], lsems[s])
        pltpu.async_copy(cw_hbm.at[b, p, :, pl.ds(pt0, PCH)],
                         wbuf.at[s], lsems[s])

    for kb in range(6):
        b = 2 * cid + (kb // 3)
        p = kb % 3

        zds = []
        for z in range(rows_per_tile // ZR):
            zds.append(pltpu.async_copy(
                zbuf, shared.at[pl.ds(sid * rows_per_tile + z * ZR, ZR)],
                zsem))
        for zd in zds:
            zd.wait()
        plsc.subcore_barrier()

        issue_loads(b, p, 0, 0)
        issue_loads(b, p, 1, 1)

        def drain_scatters(s):
            for c in range(4):
                pltpu.make_async_copy(valbuf.at[s, pl.ds(c * PCH, PCH)],
                                      shared.at[ibuf.at[s, c]],
                                      ssem).wait()

        def drain_loads(j, s):
            pt0 = sid * pts_per_tile + j * PCH
            pltpu.make_async_copy(feat_hbm.at[b, pl.ds(pt0, PCH)],
                                  fbuf.at[s], lsems[s]).wait()
            pltpu.make_async_copy(cidx_hbm.at[b, p, :, pl.ds(pt0, PCH)],
                                  ibuf.at[lax.rem(j, 4)], lsems[s]).wait()
            pltpu.make_async_copy(cw_hbm.at[b, p, :, pl.ds(pt0, PCH)],
                                  wbuf.at[s], lsems[s]).wait()

        def pipe2(j2, _):
            for s in range(2):
                j = 2 * j2 + s
                jm4 = lax.rem(j, 4)
                drain_loads(j, s)

                @pl.when(j2 > 0)
                def _():
                    drain_scatters(s)

                def q_body(q, _):
                    g = q // 16
                    l = q - g * 16
                    f = [fbuf[s, q, pl.ds(16 * t, 16)]
                         for t in range(C // 16)]
                    for c in range(4):
                        wg = wbuf[s, c, pl.ds(g * 16, 16)]
                        wv = _lane_bcast(wg, l)
                        row = c * PCH + q
                        for t in range(C // 16):
                            valbuf[s, row, pl.ds(16 * t, 16)] = f[t] * wv
                        valbuf[s, row, pl.ds(C, 16)] = wv
                    return 0
                lax.fori_loop(0, PCH, q_body, 0, unroll=2)

                for c in range(4):
                    pltpu.async_copy(valbuf.at[s, pl.ds(c * PCH, PCH)],
                                     shared.at[ibuf.at[jm4, c]], ssem,
                                     add=True)

                issue_loads(b, p, lax.rem(j + 2, nchunk), s)
            return 0
        lax.fori_loop(0, nchunk // 2, pipe2, 0)
        drain_scatters(0)
        drain_scatters(1)
        drain_loads(0, 0)
        drain_loads(1, 1)
        plsc.subcore_barrier()

        NB = nbuf.shape[0]
        def norm_body(r, _):
            row0 = sid * rows_per_tile + r * NB
            pltpu.async_copy(shared.at[pl.ds(row0, NB)], nbuf, nsem).wait()

            def rbody(i, _):
                w = nbuf[i, pl.ds(C, 16)]
                rec = jnp.float32(1.0) / jnp.maximum(w, jnp.float32(1e-6))
                for t in range(C // 16):
                    obuf[i, pl.ds(16 * t, 16)] = nbuf[i, pl.ds(16 * t, 16)] * rec
                return 0
            lax.fori_loop(0, NB, rbody, 0)
            pltpu.sync_copy(obuf, planes_hbm.at[b, p, pl.ds(row0, NB)])
            return 0
        lax.fori_loop(0, rows_per_tile // NB, norm_body, 0)
        plsc.subcore_barrier()


def _splat(ctx_feat, cidx, cw):
    B = ctx_feat.shape[0]
    mesh = plsc.VectorSubcoreMesh(core_axis_name="c", subcore_axis_name="s")
    return pl.kernel(
        _splat_body,
        compiler_params=_SC_PARAMS,
        out_type=jax.ShapeDtypeStruct((B, 3, NROWS, C), jnp.float32),
        mesh=mesh,
        scratch_types=[
            pltpu.VMEM_SHARED((NROWS, ROWW), jnp.float32),
            pltpu.VMEM((32, ROWW), jnp.float32),
            pltpu.VMEM((2, PCH, C), jnp.float32),
            pltpu.VMEM((4, 4, PCH), jnp.int32),
            pltpu.VMEM((2, 4, PCH), jnp.float32),
            pltpu.VMEM((2, 4 * PCH, ROWW), jnp.float32),
            pltpu.VMEM((128, ROWW), jnp.float32),
            pltpu.VMEM((128, C), jnp.float32),
            pltpu.SemaphoreType.DMA,
            pltpu.SemaphoreType.DMA,
            pltpu.SemaphoreType.DMA,
            pltpu.SemaphoreType.DMA,
            pltpu.SemaphoreType.DMA,
        ],
    )(ctx_feat, cidx, cw)


def _sample_body(planes_hbm, qidx_hbm, qw_hbm, fused_hbm,
                 ibuf, wbuf, rbuf, obuf, gsem0, gsem1, isem):
    cid = lax.axis_index("c")
    sid = lax.axis_index("s")
    wid = sid * NC + cid
    BQ = fused_hbm.shape[0]
    q_per_tile = BQ // (NC * NS)
    nchunk = q_per_tile // QCH
    gsems = [gsem0, gsem1]

    def issue_idx(j):
        qb = wid * q_per_tile + j * QCH
        r = lax.rem(j, 4)
        pltpu.async_copy(qidx_hbm.at[:, :, pl.ds(qb, QCH)], ibuf.at[r],
                         isem)
        pltpu.async_copy(qw_hbm.at[:, :, pl.ds(qb, QCH)], wbuf.at[r], isem)

    def drain_idx(j):
        qb = wid * q_per_tile + j * QCH
        r = lax.rem(j, 4)
        pltpu.make_async_copy(qidx_hbm.at[:, :, pl.ds(qb, QCH)],
                              ibuf.at[r], isem).wait()
        pltpu.make_async_copy(qw_hbm.at[:, :, pl.ds(qb, QCH)],
                              wbuf.at[r], isem).wait()

    def fire_gathers(j, s):
        r = lax.rem(j, 4)
        for p in range(3):
            for c in range(4):
                pltpu.async_copy(planes_hbm.at[ibuf.at[r, p, c]],
                                 rbuf.at[s, 4 * p + c], gsems[s])

    def drain_gathers(j, s):
        r = lax.rem(j, 4)
        for p in range(3):
            for c in range(4):
                pltpu.make_async_copy(planes_hbm.at[ibuf.at[r, p, c]],
                                      rbuf.at[s, 4 * p + c],
                                      gsems[s]).wait()

    for j in range(4):
        issue_idx(j)
    drain_idx(0)
    fire_gathers(0, 0)
    drain_idx(1)
    fire_gathers(1, 1)

    def pipe2(j2, _):
        for s in range(2):
            j = 2 * j2 + s
            drain_gathers(j, s)

            def q_body(q, _):
                g = q // 16
                l = q - g * 16
                r = lax.rem(j, 4)
                acc = []
                for p in range(3):
                    a = None
                    for c in range(4):
                        wg = wbuf[r, p, c, pl.ds(g * 16, 16)]
                        wv = _lane_bcast(wg, l)
                        t4 = [rbuf[s, 4 * p + c, q, pl.ds(16 * t, 16)] * wv
                              for t in range(C // 16)]
                        a = t4 if a is None else [x + y
                                                  for x, y in zip(a, t4)]
                    acc.append(a)
                for t in range(C // 16):
                    obuf[q, pl.ds(16 * t, 16)] = (acc[0][t] * acc[1][t]
                                                  * acc[2][t])
                return 0
            lax.fori_loop(0, QCH, q_body, 0)
            qb = wid * q_per_tile + j * QCH
            pltpu.sync_copy(obuf, fused_hbm.at[pl.ds(qb, QCH)])
            issue_idx(lax.rem(j + 4, nchunk))
            drain_idx(lax.rem(j + 2, nchunk))
            fire_gathers(lax.rem(j + 2, nchunk), s)
        return 0
    lax.fori_loop(0, nchunk // 2, pipe2, 0)
    drain_gathers(0, 0)
    drain_gathers(1, 1)
    drain_idx(2)
    drain_idx(3)


def _sample(planes_flat, qidx, qw, BQ):
    mesh = plsc.VectorSubcoreMesh(core_axis_name="c", subcore_axis_name="s")
    return pl.kernel(
        _sample_body,
        compiler_params=_SC_PARAMS,
        out_type=jax.ShapeDtypeStruct((BQ, C), jnp.float32),
        mesh=mesh,
        scratch_types=[
            pltpu.VMEM((4, 3, 4, QCH), jnp.int32),
            pltpu.VMEM((4, 3, 4, QCH), jnp.float32),
            pltpu.VMEM((2, 12, QCH, C), jnp.float32),
            pltpu.VMEM((QCH, C), jnp.float32),
            pltpu.SemaphoreType.DMA,
            pltpu.SemaphoreType.DMA,
            pltpu.SemaphoreType.DMA,
        ],
    )(planes_flat, qidx, qw)


def _qmlp_body(fused_ref, Wq1_ref, bq1_ref, Wq2_ref, bq2_ref, Wd_ref, bd_ref,
               out_ref):
    x = fused_ref[...]
    h1 = _gelu(jnp.dot(x, Wq1_ref[...],
                       preferred_element_type=jnp.float32) + bq1_ref[...])
    h2 = _gelu(jnp.dot(h1, Wq2_ref[...],
                       preferred_element_type=jnp.float32) + bq2_ref[...])
    o = jnp.sum(h2 * Wd_ref[...].reshape(1, H), axis=1, keepdims=True)
    out_ref[...] = o + bd_ref[...]


def _query_mlp(fused, Wq1, bq1, Wq2, bq2, Wd, bd):
    BQ = fused.shape[0]
    blk = 4096
    return pl.pallas_call(
        _qmlp_body,
        grid=(BQ // blk,),
        in_specs=[
            pl.BlockSpec((blk, C), lambda i: (i, 0)),
            pl.BlockSpec((C, H), lambda i: (0, 0)),
            pl.BlockSpec((H,), lambda i: (0,)),
            pl.BlockSpec((H, H), lambda i: (0, 0)),
            pl.BlockSpec((H,), lambda i: (0,)),
            pl.BlockSpec((H, 1), lambda i: (0, 0)),
            pl.BlockSpec((1,), lambda i: (0,)),
        ],
        out_specs=pl.BlockSpec((blk, 1), lambda i: (i, 0)),
        out_shape=jax.ShapeDtypeStruct((BQ, 1), jnp.float32),
    )(fused, Wq1, bq1, Wq2, bq2, Wd, bd)


def kernel(ctx_xyz, ctx_dist, qry_xyz, W1, b1, W2, b2, Wq1, bq1, Wq2, bq2,
           Wd, bd):
    B, N, _ = ctx_xyz.shape
    Q = qry_xyz.shape[1]
    BQ = B * Q

    ctx_feat, cidx, cw = _ctx_mlp(ctx_xyz, ctx_dist, W1, b1, W2, b2)
    planes = _splat(ctx_feat, cidx, cw)
    qidx, qw = _query_prep(qry_xyz.reshape(BQ, 3).T)
    fused = _sample(planes.reshape(B * 3 * NROWS, C), qidx, qw, BQ)
    out = _query_mlp(fused, Wq1, bq1, Wq2, bq2, Wd, bd)
    return out.reshape(B, Q, 1)
